# Initial kernel scaffold; baseline (speedup 1.0000x reference)
import jax
import jax.numpy as jnp
from jax.experimental import pallas as pl


def kernel(x, edge_index, W1, b1, gamma1, beta1, W2, b2):
    raise NotImplementedError("write your pallas kernel here")


# speedup vs baseline: 21.4843x; 21.4843x over previous
import functools

import jax
import jax.numpy as jnp
from jax import lax
from jax.experimental import pallas as pl
from jax.experimental.pallas import tpu as pltpu
from jax.experimental.pallas import tpu_sc as plsc

NC, NS = 2, 16
NW = NC * NS
W = 128
N_PAD = 10240


def _mesh():
    return plsc.VectorSubcoreMesh(
        core_axis_name="c", subcore_axis_name="s", num_cores=NC, num_subcores=NS
    )


def _deg_body(n_pad, ept, dst_hbm, zeros_hbm, p0, p1, dst_v, ones_v, deg_sh, sem):
    c = lax.axis_index("c")
    s = lax.axis_index("s")
    wid = c * NS + s
    rpt = n_pad // NS
    r0 = s * rpt
    for i in range(W // 16):
        ones_v[pl.ds(i * 16, 16)] = jnp.ones((16,), jnp.float32)
    pltpu.sync_copy(zeros_hbm.at[pl.ds(r0, rpt)], deg_sh.at[pl.ds(r0, rpt)])
    plsc.subcore_barrier()
    base = wid * ept

    def body(w, carry):
        off = pl.multiple_of(base + w * W, W)
        pltpu.sync_copy(dst_hbm.at[pl.ds(off, W)], dst_v)
        pltpu.async_copy(ones_v, deg_sh.at[dst_v], sem, add=True).wait()
        return carry

    lax.fori_loop(0, ept // W, body, 0)
    plsc.subcore_barrier()

    @pl.when(c == 0)
    def _():
        pltpu.sync_copy(deg_sh.at[pl.ds(r0, rpt)], p0.at[pl.ds(r0, rpt)])

    @pl.when(c == 1)
    def _():
        pltpu.sync_copy(deg_sh.at[pl.ds(r0, rpt)], p1.at[pl.ds(r0, rpt)])


def _deg_call(dst_pad, zeros_col, n_pad, ept):
    kfn = pl.kernel(
        functools.partial(_deg_body, n_pad, ept),
        out_type=(
            jax.ShapeDtypeStruct((n_pad,), jnp.float32),
            jax.ShapeDtypeStruct((n_pad,), jnp.float32),
        ),
        mesh=_mesh(),
        scratch_types=(
            pltpu.VMEM((W,), jnp.int32),
            pltpu.VMEM((W,), jnp.float32),
            pltpu.VMEM_SHARED((n_pad,), jnp.float32),
            pltpu.SemaphoreType.DMA,
        ),
        compiler_params=pltpu.CompilerParams(use_tc_tiling_on_sc=False),
    )
    return kfn(dst_pad, zeros_col)


def _prop_body(n_pad, ept, d,
               g_hbm, src_hbm, dst_hbm, zeros_hbm, p0, p1,
               src_v, dst_v, rows_v, g_sh, acc_sh, gsem, ssem):
    c = lax.axis_index("c")
    s = lax.axis_index("s")
    wid = c * NS + s
    rpt_a = n_pad // NS
    pltpu.sync_copy(g_hbm.at[pl.ds(s * rpt_a, rpt_a)],
                    g_sh.at[pl.ds(s * rpt_a, rpt_a)])
    pltpu.sync_copy(zeros_hbm.at[pl.ds(s * rpt_a, rpt_a)],
                    acc_sh.at[pl.ds(s * rpt_a, rpt_a)])
    plsc.subcore_barrier()
    base = wid * ept

    def body(w, carry):
        off = pl.multiple_of(base + w * W, W)
        pltpu.sync_copy(src_hbm.at[pl.ds(off, W)], src_v)
        pltpu.sync_copy(dst_hbm.at[pl.ds(off, W)], dst_v)
        pltpu.async_copy(g_sh.at[src_v], rows_v, gsem).wait()
        pltpu.async_copy(rows_v, acc_sh.at[dst_v], ssem, add=True).wait()
        return carry

    lax.fori_loop(0, ept // W, body, 0)
    plsc.subcore_barrier()

    @pl.when(c == 0)
    def _():
        pltpu.sync_copy(acc_sh.at[pl.ds(s * rpt_a, rpt_a)],
                        p0.at[pl.ds(s * rpt_a, rpt_a)])

    @pl.when(c == 1)
    def _():
        pltpu.sync_copy(acc_sh.at[pl.ds(s * rpt_a, rpt_a)],
                        p1.at[pl.ds(s * rpt_a, rpt_a)])


def _prop_call(g_padded, src_pad, dst_pad, zeros_nd, n_pad, ept, d):
    kfn = pl.kernel(
        functools.partial(_prop_body, n_pad, ept, d),
        out_type=(
            jax.ShapeDtypeStruct((n_pad, d), jnp.float32),
            jax.ShapeDtypeStruct((n_pad, d), jnp.float32),
        ),
        mesh=_mesh(),
        scratch_types=(
            pltpu.VMEM((W,), jnp.int32),
            pltpu.VMEM((W,), jnp.int32),
            pltpu.VMEM((W, d), jnp.float32),
            pltpu.VMEM_SHARED((n_pad, d), jnp.float32),
            pltpu.VMEM_SHARED((n_pad, d), jnp.float32),
            pltpu.SemaphoreType.DMA,
            pltpu.SemaphoreType.DMA,
        ),
        compiler_params=pltpu.CompilerParams(use_tc_tiling_on_sc=False),
    )
    return kfn(g_padded, src_pad, dst_pad, zeros_nd)


def _tc1_body(x_ref, w1_ref, p0_ref, p1_ref, g1_ref, dinv_ref):
    deg = p0_ref[...] + p1_ref[...] + 1.0
    dinv = lax.rsqrt(deg)
    dinv_ref[...] = dinv
    h1 = jnp.dot(x_ref[...], w1_ref[...], preferred_element_type=jnp.float32)
    g1_ref[...] = h1 * dinv


def _tc2_body(a0_ref, a1_ref, g1_ref, dinv_ref, gamma_ref, beta_ref, w2_ref,
              g2_ref):
    dinv = dinv_ref[...]
    h = (a0_ref[...] + a1_ref[...] + g1_ref[...]) * dinv
    mu = jnp.mean(h, axis=0, keepdims=True)
    var = jnp.mean((h - mu) * (h - mu), axis=0, keepdims=True)
    hn = (h - mu) * lax.rsqrt(var + 1e-5) * gamma_ref[...] + beta_ref[...]
    hr = jnp.maximum(hn, 0.0)
    h2 = jnp.dot(hr, w2_ref[...], preferred_element_type=jnp.float32)
    g2_ref[...] = h2 * dinv


def _tc3_body(a0_ref, a1_ref, g2_ref, dinv_ref, b2_ref, out_ref):
    out_ref[...] = (a0_ref[...] + a1_ref[...] + g2_ref[...]) * dinv_ref[...] \
        + b2_ref[...]


def _tc_call(body, out_shapes, *args):
    return pl.pallas_call(
        body,
        out_shape=out_shapes,
    )(*args)


def kernel(x, edge_index, W1, b1, gamma1, beta1, W2, b2):
    n = x.shape[0]
    e = edge_index.shape[1]
    d_hid = W1.shape[1]
    d_out = W2.shape[1]

    src = edge_index[0].astype(jnp.int32)
    dst = edge_index[1].astype(jnp.int32)
    ept = ((e + NW * W - 1) // (NW * W)) * W
    e_pad = ept * NW
    src_pad = jnp.concatenate([src, jnp.zeros((e_pad - e,), jnp.int32)])
    dst_pad = jnp.concatenate([dst, jnp.full((e_pad - e,), n, jnp.int32)])

    zeros_nd = jnp.zeros((N_PAD, max(d_hid, d_out)), jnp.float32)
    zeros_col = jnp.zeros((N_PAD,), jnp.float32)

    dp0, dp1 = _deg_call(dst_pad, zeros_col, N_PAD, ept)

    g1, dinv = _tc_call(
        _tc1_body,
        (jax.ShapeDtypeStruct((n, d_hid), jnp.float32),
         jax.ShapeDtypeStruct((n, 1), jnp.float32)),
        x, W1, dp0[:n].reshape(n, 1), dp1[:n].reshape(n, 1),
    )

    g1_pad = jnp.concatenate(
        [g1, jnp.zeros((N_PAD - n, d_hid), jnp.float32)])
    a0, a1 = _prop_call(g1_pad, src_pad, dst_pad, zeros_nd[:, :d_hid],
                        N_PAD, ept, d_hid)

    g2 = _tc_call(
        _tc2_body,
        jax.ShapeDtypeStruct((n, d_out), jnp.float32),
        a0[:n], a1[:n], g1, dinv,
        gamma1.reshape(1, d_hid), beta1.reshape(1, d_hid), W2,
    )

    g2_pad = jnp.concatenate(
        [g2, jnp.zeros((N_PAD - n, d_out), jnp.float32)])
    b0, b1_ = _prop_call(g2_pad, src_pad, dst_pad, zeros_nd[:, :d_out],
                         N_PAD, ept, d_out)

    out = _tc_call(
        _tc3_body,
        jax.ShapeDtypeStruct((n, d_out), jnp.float32),
        b0[:n], b1_[:n], g2, dinv, b2.reshape(1, d_out),
    )
    return out


# speedup vs baseline: 35.6330x; 1.6586x over previous
import functools

import jax
import jax.numpy as jnp
from jax import lax
from jax.experimental import pallas as pl
from jax.experimental.pallas import tpu as pltpu
from jax.experimental.pallas import tpu_sc as plsc

NC, NS = 2, 16
NW = NC * NS
W = 512
N_PAD = 10240


def _mesh():
    return plsc.VectorSubcoreMesh(
        core_axis_name="c", subcore_axis_name="s", num_cores=NC, num_subcores=NS
    )


def _worker_windows(nwin):
    c = lax.axis_index("c")
    s = lax.axis_index("s")
    wid = c * NS + s
    lo = wid * nwin // NW
    hi = (wid + 1) * nwin // NW
    return s, lo, hi


def _deg_body(n_pad, nwin, ei_hbm, zeros_hbm, p0, p1, dst_v, ones_v, deg_sh,
              isem, ssem):
    c = lax.axis_index("c")
    s, lo, hi = _worker_windows(nwin)
    rpt = n_pad // NS
    r0 = s * rpt
    for i in range(W // 16):
        ones_v[pl.ds(i * 16, 16)] = jnp.ones((16,), jnp.float32)
    pltpu.sync_copy(zeros_hbm.at[pl.ds(r0, rpt)], deg_sh.at[pl.ds(r0, rpt)])
    plsc.subcore_barrier()

    def body(w, carry):
        off = pl.multiple_of(w * W, W)
        pltpu.sync_copy(ei_hbm.at[1, pl.ds(off, W)], dst_v)
        pltpu.async_copy(ones_v, deg_sh.at[dst_v], ssem, add=True).wait()
        return carry

    lax.fori_loop(lo, hi, body, 0)
    plsc.subcore_barrier()

    @pl.when(c == 0)
    def _():
        pltpu.sync_copy(deg_sh.at[pl.ds(r0, rpt)], p0.at[pl.ds(r0, rpt)])

    @pl.when(c == 1)
    def _():
        pltpu.sync_copy(deg_sh.at[pl.ds(r0, rpt)], p1.at[pl.ds(r0, rpt)])


def _deg_call(ei, zeros_col, n_pad, nwin):
    kfn = pl.kernel(
        functools.partial(_deg_body, n_pad, nwin),
        out_type=(
            jax.ShapeDtypeStruct((n_pad,), jnp.float32),
            jax.ShapeDtypeStruct((n_pad,), jnp.float32),
        ),
        mesh=_mesh(),
        scratch_types=(
            pltpu.VMEM((W,), jnp.int32),
            pltpu.VMEM((W,), jnp.float32),
            pltpu.VMEM_SHARED((n_pad,), jnp.float32),
            pltpu.SemaphoreType.DMA,
            pltpu.SemaphoreType.DMA,
        ),
        compiler_params=pltpu.CompilerParams(use_tc_tiling_on_sc=False),
    )
    return kfn(ei, zeros_col)


def _prop_body(n_pad, nwin, d,
               g_hbm, ei_hbm, zeros_hbm, p0, p1,
               src_v, dst_v, rows_v, g_sh, acc_sh, isem, gsem, ssem):
    c = lax.axis_index("c")
    s, lo, hi = _worker_windows(nwin)
    rpt = n_pad // NS
    r0 = s * rpt
    pltpu.sync_copy(g_hbm.at[pl.ds(r0, rpt)], g_sh.at[pl.ds(r0, rpt)])
    pltpu.sync_copy(zeros_hbm.at[pl.ds(r0, rpt)], acc_sh.at[pl.ds(r0, rpt)])
    plsc.subcore_barrier()

    def body(w, carry):
        off = pl.multiple_of(w * W, W)
        pltpu.sync_copy(ei_hbm.at[0, pl.ds(off, W)], src_v)
        pltpu.sync_copy(ei_hbm.at[1, pl.ds(off, W)], dst_v)
        pltpu.async_copy(g_sh.at[src_v], rows_v, gsem).wait()
        pltpu.async_copy(rows_v, acc_sh.at[dst_v], ssem, add=True).wait()
        return carry

    lax.fori_loop(lo, hi, body, 0)
    plsc.subcore_barrier()

    @pl.when(c == 0)
    def _():
        pltpu.sync_copy(acc_sh.at[pl.ds(r0, rpt)], p0.at[pl.ds(r0, rpt)])

    @pl.when(c == 1)
    def _():
        pltpu.sync_copy(acc_sh.at[pl.ds(r0, rpt)], p1.at[pl.ds(r0, rpt)])


def _prop_call(g_padded, ei, zeros_nd, n_pad, nwin, d):
    kfn = pl.kernel(
        functools.partial(_prop_body, n_pad, nwin, d),
        out_type=(
            jax.ShapeDtypeStruct((n_pad, d), jnp.float32),
            jax.ShapeDtypeStruct((n_pad, d), jnp.float32),
        ),
        mesh=_mesh(),
        scratch_types=(
            pltpu.VMEM((W,), jnp.int32),
            pltpu.VMEM((W,), jnp.int32),
            pltpu.VMEM((W, d), jnp.float32),
            pltpu.VMEM_SHARED((n_pad, d), jnp.float32),
            pltpu.VMEM_SHARED((n_pad, d), jnp.float32),
            pltpu.SemaphoreType.DMA,
            pltpu.SemaphoreType.DMA,
            pltpu.SemaphoreType.DMA,
        ),
        compiler_params=pltpu.CompilerParams(use_tc_tiling_on_sc=False),
    )
    return kfn(g_padded, ei, zeros_nd)


def _tc1_body(n, x_ref, w1_ref, p0_ref, p1_ref, g1_ref, dinv_ref):
    deg = p0_ref[pl.ds(0, n)] + p1_ref[pl.ds(0, n)] + 1.0
    dinv = lax.rsqrt(deg)
    dinv_ref[...] = dinv
    h1 = jnp.dot(x_ref[...], w1_ref[...], preferred_element_type=jnp.float32)
    g1_ref[pl.ds(0, n), :] = h1 * dinv
    g1_ref[pl.ds(n, g1_ref.shape[0] - n), :] = jnp.zeros(
        (g1_ref.shape[0] - n, g1_ref.shape[1]), jnp.float32)


def _tc2_body(n, a0_ref, a1_ref, g1_ref, dinv_ref, gamma_ref, beta_ref,
              w2_ref, g2_ref):
    dinv = dinv_ref[...]
    h = (a0_ref[pl.ds(0, n)] + a1_ref[pl.ds(0, n)] + g1_ref[pl.ds(0, n)]) \
        * dinv
    mu = jnp.mean(h, axis=0, keepdims=True)
    var = jnp.mean((h - mu) * (h - mu), axis=0, keepdims=True)
    hn = (h - mu) * lax.rsqrt(var + 1e-5) * gamma_ref[...] + beta_ref[...]
    hr = jnp.maximum(hn, 0.0)
    h2 = jnp.dot(hr, w2_ref[...], preferred_element_type=jnp.float32)
    g2_ref[pl.ds(0, n), :] = h2 * dinv
    g2_ref[pl.ds(n, g2_ref.shape[0] - n), :] = jnp.zeros(
        (g2_ref.shape[0] - n, g2_ref.shape[1]), jnp.float32)


def _tc3_body(n, a0_ref, a1_ref, g2_ref, dinv_ref, b2_ref, out_ref):
    out_ref[...] = (a0_ref[pl.ds(0, n)] + a1_ref[pl.ds(0, n)]
                    + g2_ref[pl.ds(0, n)]) * dinv_ref[...] + b2_ref[...]


def _tc_call(body, out_shapes, *args):
    return pl.pallas_call(body, out_shape=out_shapes)(*args)


def kernel(x, edge_index, W1, b1, gamma1, beta1, W2, b2):
    n = x.shape[0]
    e = edge_index.shape[1]
    d_hid = W1.shape[1]
    d_out = W2.shape[1]
    assert e % W == 0, "edge count must be window-aligned"
    nwin = e // W

    ei = edge_index.astype(jnp.int32)

    zeros_nd = jnp.zeros((N_PAD, max(d_hid, d_out)), jnp.float32)
    zeros_col = jnp.zeros((N_PAD,), jnp.float32)

    dp0, dp1 = _deg_call(ei, zeros_col, N_PAD, nwin)
    dp0 = dp0.reshape(N_PAD, 1)
    dp1 = dp1.reshape(N_PAD, 1)

    g1, dinv = _tc_call(
        functools.partial(_tc1_body, n),
        (jax.ShapeDtypeStruct((N_PAD, d_hid), jnp.float32),
         jax.ShapeDtypeStruct((n, 1), jnp.float32)),
        x, W1, dp0, dp1,
    )

    a0, a1 = _prop_call(g1, ei, zeros_nd[:, :d_hid], N_PAD, nwin, d_hid)

    g2 = _tc_call(
        functools.partial(_tc2_body, n),
        jax.ShapeDtypeStruct((N_PAD, d_out), jnp.float32),
        a0, a1, g1, dinv,
        gamma1.reshape(1, d_hid), beta1.reshape(1, d_hid), W2,
    )

    b0, b1_ = _prop_call(g2, ei, zeros_nd[:, :d_out], N_PAD, nwin, d_out)

    out = _tc_call(
        functools.partial(_tc3_body, n),
        jax.ShapeDtypeStruct((n, d_out), jnp.float32),
        b0, b1_, g2, dinv, b2.reshape(1, d_out),
    )
    return out


# speedup vs baseline: 43.8859x; 1.2316x over previous
import functools

import jax
import jax.numpy as jnp
from jax import lax
from jax.experimental import pallas as pl
from jax.experimental.pallas import tpu as pltpu
from jax.experimental.pallas import tpu_sc as plsc

NC, NS = 2, 16
NW = NC * NS
N_PAD = 10240
W_DEG = 2000
W_PROP = 400


def _mesh():
    return plsc.VectorSubcoreMesh(
        core_axis_name="c", subcore_axis_name="s", num_cores=NC, num_subcores=NS
    )


def _params():
    return pltpu.CompilerParams(use_tc_tiling_on_sc=False)


def _deg_body(n_pad, nwin, ei_hbm, zeros_hbm, p0, p1,
              dst_v, ones_v, deg_sh, isem, ssem):
    c = lax.axis_index("c")
    s = lax.axis_index("s")
    wid = c * NS + s
    rpt = n_pad // NS
    r0 = s * rpt
    for i in range(W_DEG // 16):
        ones_v[pl.ds(i * 16, 16)] = jnp.ones((16,), jnp.float32)
    pltpu.sync_copy(zeros_hbm.at[pl.ds(r0, rpt)], deg_sh.at[pl.ds(r0, rpt)])
    plsc.subcore_barrier()

    base = wid * nwin
    idx_d = [None] * nwin
    sc_d = [None] * nwin
    for w in range(min(2, nwin)):
        idx_d[w] = pltpu.async_copy(
            ei_hbm.at[1, pl.ds((base + w) * W_DEG, W_DEG)], dst_v[w % 4], isem)
    for w in range(nwin):
        idx_d[w].wait()
        if w >= 2:
            sc_d[w - 2].wait()
        sc_d[w] = pltpu.async_copy(
            ones_v, deg_sh.at[dst_v[w % 4]], ssem, add=True)
        if w + 2 < nwin:
            idx_d[w + 2] = pltpu.async_copy(
                ei_hbm.at[1, pl.ds((base + w + 2) * W_DEG, W_DEG)],
                dst_v[(w + 2) % 4], isem)
    for w in range(max(0, nwin - 2), nwin):
        sc_d[w].wait()
    plsc.subcore_barrier()

    @pl.when(c == 0)
    def _():
        pltpu.sync_copy(deg_sh.at[pl.ds(r0, rpt)], p0.at[pl.ds(r0, rpt)])

    @pl.when(c == 1)
    def _():
        pltpu.sync_copy(deg_sh.at[pl.ds(r0, rpt)], p1.at[pl.ds(r0, rpt)])


def _deg_call(ei, zeros_col, n_pad, nwin):
    kfn = pl.kernel(
        functools.partial(_deg_body, n_pad, nwin),
        out_type=(
            jax.ShapeDtypeStruct((n_pad,), jnp.float32),
            jax.ShapeDtypeStruct((n_pad,), jnp.float32),
        ),
        mesh=_mesh(),
        scratch_types=(
            [pltpu.VMEM((W_DEG,), jnp.int32) for _ in range(4)],
            pltpu.VMEM((W_DEG,), jnp.float32),
            pltpu.VMEM_SHARED((n_pad,), jnp.float32),
            pltpu.SemaphoreType.DMA,
            pltpu.SemaphoreType.DMA,
        ),
        compiler_params=_params(),
    )
    return kfn(ei, zeros_col)


def _prop_body(n_pad, nwin, d,
               g_hbm, ei_hbm, zeros_hbm, p0, p1,
               src_v, dst_v, rows_v, acc_sh, isem, gsem, ssem):
    c = lax.axis_index("c")
    s = lax.axis_index("s")
    wid = c * NS + s
    rpt = n_pad // NS
    r0 = s * rpt
    pltpu.sync_copy(zeros_hbm.at[pl.ds(r0, rpt)], acc_sh.at[pl.ds(r0, rpt)])
    plsc.subcore_barrier()

    W = W_PROP
    base = wid * nwin
    idx_d = [None] * nwin
    g_d = [None] * nwin
    sc_d = [None] * nwin

    def start_idx(w):
        idx_d[w] = (
            pltpu.async_copy(
                ei_hbm.at[0, pl.ds((base + w) * W, W)], src_v[w % 3], isem),
            pltpu.async_copy(
                ei_hbm.at[1, pl.ds((base + w) * W, W)], dst_v[w % 3], isem),
        )

    for w in range(min(2, nwin)):
        start_idx(w)
    idx_d[0][0].wait()
    idx_d[0][1].wait()
    g_d[0] = pltpu.async_copy(g_hbm.at[src_v[0]], rows_v[0], gsem)

    for w in range(nwin):
        b = w % 2
        g_d[w].wait()
        if w >= 1:
            sc_d[w - 1].wait()
        if w + 1 < nwin:
            idx_d[w + 1][0].wait()
            idx_d[w + 1][1].wait()
            g_d[w + 1] = pltpu.async_copy(
                g_hbm.at[src_v[(w + 1) % 3]], rows_v[(w + 1) % 2], gsem)
        sc_d[w] = pltpu.async_copy(
            rows_v[b], acc_sh.at[dst_v[w % 3]], ssem, add=True)
        if w + 2 < nwin:
            start_idx(w + 2)
    sc_d[nwin - 1].wait()
    plsc.subcore_barrier()

    @pl.when(c == 0)
    def _():
        pltpu.sync_copy(acc_sh.at[pl.ds(r0, rpt)], p0.at[pl.ds(r0, rpt)])

    @pl.when(c == 1)
    def _():
        pltpu.sync_copy(acc_sh.at[pl.ds(r0, rpt)], p1.at[pl.ds(r0, rpt)])


def _prop_call(g, ei, zeros_nd, n_pad, nwin, d):
    kfn = pl.kernel(
        functools.partial(_prop_body, n_pad, nwin, d),
        out_type=(
            jax.ShapeDtypeStruct((n_pad, d), jnp.float32),
            jax.ShapeDtypeStruct((n_pad, d), jnp.float32),
        ),
        mesh=_mesh(),
        scratch_types=(
            [pltpu.VMEM((W_PROP,), jnp.int32) for _ in range(3)],
            [pltpu.VMEM((W_PROP,), jnp.int32) for _ in range(3)],
            [pltpu.VMEM((W_PROP, d), jnp.float32) for _ in range(2)],
            pltpu.VMEM_SHARED((n_pad, d), jnp.float32),
            pltpu.SemaphoreType.DMA,
            pltpu.SemaphoreType.DMA,
            pltpu.SemaphoreType.DMA,
        ),
        compiler_params=_params(),
    )
    return kfn(g, ei, zeros_nd)


def _tc1_body(x_ref, w1_ref, p0_ref, p1_ref, g1_ref, dinv_ref):
    deg = p0_ref[...] + p1_ref[...] + 1.0
    dinv = lax.rsqrt(deg)
    dinv_ref[...] = dinv
    h1 = jnp.dot(x_ref[...], w1_ref[...], preferred_element_type=jnp.float32)
    g1_ref[...] = h1 * dinv


def _tc2_body(n, a0_ref, a1_ref, g1_ref, dinv_ref, gamma_ref, beta_ref,
              w2_ref, g2_ref):
    dinv = dinv_ref[...]
    h = (a0_ref[pl.ds(0, n)] + a1_ref[pl.ds(0, n)] + g1_ref[...]) * dinv
    mu = jnp.mean(h, axis=0, keepdims=True)
    var = jnp.mean((h - mu) * (h - mu), axis=0, keepdims=True)
    hn = (h - mu) * lax.rsqrt(var + 1e-5) * gamma_ref[...] + beta_ref[...]
    hr = jnp.maximum(hn, 0.0)
    h2 = jnp.dot(hr, w2_ref[...], preferred_element_type=jnp.float32)
    g2_ref[...] = h2 * dinv


def _tc3_body(n, a0_ref, a1_ref, g2_ref, dinv_ref, b2_ref, out_ref):
    out_ref[...] = (a0_ref[pl.ds(0, n)] + a1_ref[pl.ds(0, n)]
                    + g2_ref[...]) * dinv_ref[...] + b2_ref[...]


def _tc_call(body, out_shapes, *args):
    return pl.pallas_call(body, out_shape=out_shapes)(*args)


def kernel(x, edge_index, W1, b1, gamma1, beta1, W2, b2):
    n = x.shape[0]
    e = edge_index.shape[1]
    d_hid = W1.shape[1]
    d_out = W2.shape[1]
    assert e % (NW * W_DEG) == 0 and e % (NW * W_PROP) == 0

    ei = edge_index.astype(jnp.int32)

    zeros_nd = jnp.zeros((N_PAD, max(d_hid, d_out)), jnp.float32)
    zeros_col = jnp.zeros((N_PAD,), jnp.float32)

    dp0, dp1 = _deg_call(ei, zeros_col, N_PAD, e // (NW * W_DEG))
    dp0 = dp0.reshape(N_PAD, 1)[:n]
    dp1 = dp1.reshape(N_PAD, 1)[:n]

    g1, dinv = _tc_call(
        _tc1_body,
        (jax.ShapeDtypeStruct((n, d_hid), jnp.float32),
         jax.ShapeDtypeStruct((n, 1), jnp.float32)),
        x, W1, dp0, dp1,
    )

    a0, a1 = _prop_call(g1, ei, zeros_nd[:, :d_hid],
                        N_PAD, e // (NW * W_PROP), d_hid)

    g2 = _tc_call(
        functools.partial(_tc2_body, n),
        jax.ShapeDtypeStruct((n, d_out), jnp.float32),
        a0, a1, g1, dinv,
        gamma1.reshape(1, d_hid), beta1.reshape(1, d_hid), W2,
    )

    b0, b1_ = _prop_call(g2, ei, zeros_nd[:, :d_out],
                         N_PAD, e // (NW * W_PROP), d_out)

    out = _tc_call(
        functools.partial(_tc3_body, n),
        jax.ShapeDtypeStruct((n, d_out), jnp.float32),
        b0, b1_, g2, dinv, b2.reshape(1, d_out),
    )
    return out
